# Initial kernel scaffold; baseline (speedup 1.0000x reference)
import jax
import jax.numpy as jnp
from jax.experimental import pallas as pl


def kernel(embedding, hidden_states, last_token_indices, temperatures, top_ps):
    raise NotImplementedError("write your pallas kernel here")


# speedup vs baseline: 22.0537x; 22.0537x over previous
import functools

import jax
import jax.numpy as jnp
from jax import lax
from jax.experimental import pallas as pl
from jax.experimental.pallas import tpu as pltpu
from jax.experimental.pallas import tpu_sc as plsc

V = 100000
D = 2048
B = 64
VB = 2048
NB = (V + VB - 1) // VB
VPAD = NB * VB
BISECT_ITERS = 30
LO_BITS = 0x33000000
HI_BITS = 0x3F800000

_RPW = 8
_NW_USED = B // _RPW


def _sc_gather(table, idx):
    info = plsc.get_sparse_core_info()
    nc = info.num_cores
    mesh = plsc.VectorSubcoreMesh(core_axis_name="c", subcore_axis_name="s")

    @functools.partial(
        pl.kernel,
        out_type=jax.ShapeDtypeStruct((B, D), jnp.float32),
        mesh=mesh,
        scratch_types=[
            pltpu.VMEM((_RPW,), jnp.int32),
            pltpu.VMEM((_RPW, D), jnp.float32),
            pltpu.SemaphoreType.DMA,
        ],
    )
    def k(table_hbm, idx_hbm, out_hbm, idx_v, rows_v, sem):
        wid = lax.axis_index("s") * nc + lax.axis_index("c")

        @pl.when(wid < _NW_USED)
        def _():
            base = wid * _RPW
            pltpu.sync_copy(idx_hbm.at[pl.ds(base, _RPW)], idx_v)
            pltpu.async_copy(table_hbm.at[idx_v], rows_v, sem).wait()
            pltpu.sync_copy(rows_v, out_hbm.at[pl.ds(base, _RPW)])

    return k(table, idx)


def _sampler_body(hs_ref, emb_ref, temp_ref, top_ref, out_ref, ids_ref,
                  xs_ref, m_ref, z_ref, thr_ref, den_ref, gmax_ref, gidx_ref):
    i = pl.program_id(0)

    @pl.when(i < NB)
    def _matmul():
        eb = emb_ref[...]
        x = lax.dot_general(hs_ref[...], eb,
                            (((1,), (1,)), ((), ())),
                            preferred_element_type=jnp.float32)
        x = x / temp_ref[...]
        col = i * VB + lax.broadcasted_iota(jnp.int32, (B, VB), 1)
        x = jnp.where(col < V, x, -1e30)
        xs_ref[:, pl.ds(pl.multiple_of(i * VB, VB), VB)] = x
        bm = jnp.max(x, axis=1, keepdims=True)
        m_ref[...] = jnp.where(i == 0, bm, jnp.maximum(m_ref[...], bm))

    @pl.when(i == NB)
    def _threshold():
        m = m_ref[...]

        def expz(k, z):
            sl = pl.ds(pl.multiple_of(k * VB, VB), VB)
            u = jnp.exp(xs_ref[:, sl] - m)
            xs_ref[:, sl] = u
            return z + jnp.sum(u, axis=1, keepdims=True)

        z = lax.fori_loop(0, NB, expz, jnp.zeros((B, 1), jnp.float32))
        c = top_ref[...] * z

        def bisect(_, lohi):
            lo, hi = lohi
            mid = (lo + hi) >> 1
            t = lax.bitcast_convert_type(mid, jnp.float32)

            def s_chunk(k, s):
                u = xs_ref[:, pl.ds(pl.multiple_of(k * VB, VB), VB)]
                return s + jnp.sum(jnp.where(u > t, u, 0.0),
                                   axis=1, keepdims=True)

            s = lax.fori_loop(0, NB, s_chunk, jnp.zeros((B, 1), jnp.float32))
            pred = s <= c
            return jnp.where(pred, lo, mid), jnp.where(pred, mid, hi)

        lo0 = jnp.full((B, 1), LO_BITS, jnp.int32)
        hi0 = jnp.full((B, 1), HI_BITS, jnp.int32)
        _, hi = lax.fori_loop(0, BISECT_ITERS, bisect, (lo0, hi0))
        thr = lax.bitcast_convert_type(hi, jnp.float32)

        def kept(k, s):
            u = xs_ref[:, pl.ds(pl.multiple_of(k * VB, VB), VB)]
            return s + jnp.sum(jnp.where(u >= thr, u / z, 0.0),
                               axis=1, keepdims=True)

        den = lax.fori_loop(0, NB, kept, jnp.zeros((B, 1), jnp.float32))
        z_ref[...] = z
        thr_ref[...] = thr
        den_ref[...] = den

    @pl.when(i > NB)
    def _emit():
        blk = i - NB - 1
        u = xs_ref[:, pl.ds(pl.multiple_of(blk * VB, VB), VB)]
        p = u / z_ref[...]
        pf = jnp.where(u >= thr_ref[...], p / den_ref[...], 0.0)
        out_ref[...] = pf
        cm = jnp.max(pf, axis=1, keepdims=True)
        ci = jnp.argmax(pf, axis=1).astype(jnp.int32)[:, None] + blk * VB

        @pl.when(blk == 0)
        def _():
            gmax_ref[...] = cm
            gidx_ref[...] = ci

        @pl.when(blk > 0)
        def _():
            upd = cm > gmax_ref[...]
            gidx_ref[...] = jnp.where(upd, ci, gidx_ref[...])
            gmax_ref[...] = jnp.maximum(gmax_ref[...], cm)

        @pl.when(i == 2 * NB)
        def _():
            ids_ref[...] = gidx_ref[...]


_GRID = (2 * NB + 1,)
_IN_SPECS = [
    pl.BlockSpec((B, D), lambda i: (0, 0)),
    pl.BlockSpec((VB, D), lambda i: (jnp.minimum(i, NB - 1), 0)),
    pl.BlockSpec((B, 1), lambda i: (0, 0)),
    pl.BlockSpec((B, 1), lambda i: (0, 0)),
]
_OUT_SPECS = [
    pl.BlockSpec((B, VB), lambda i: (0, jnp.maximum(0, i - (NB + 1)))),
    pl.BlockSpec((B, 1), lambda i: (0, 0)),
]
_OUT_SHAPE = [
    jax.ShapeDtypeStruct((B, V), jnp.float32),
    jax.ShapeDtypeStruct((B, 1), jnp.int32),
]
_SCRATCH = [
    pltpu.VMEM((B, VPAD), jnp.float32),
    pltpu.VMEM((B, 1), jnp.float32),
    pltpu.VMEM((B, 1), jnp.float32),
    pltpu.VMEM((B, 1), jnp.float32),
    pltpu.VMEM((B, 1), jnp.float32),
    pltpu.VMEM((B, 1), jnp.float32),
    pltpu.VMEM((B, 1), jnp.int32),
]


def kernel(embedding, hidden_states, last_token_indices, temperatures, top_ps):
    hs = _sc_gather(hidden_states, last_token_indices.astype(jnp.int32))
    probs, ids = pl.pallas_call(
        _sampler_body,
        grid=_GRID,
        in_specs=_IN_SPECS,
        out_specs=_OUT_SPECS,
        out_shape=_OUT_SHAPE,
        scratch_shapes=_SCRATCH,
        compiler_params=pltpu.CompilerParams(vmem_limit_bytes=100 * 1024 * 1024),
    )(hs, embedding, temperatures[:, None], top_ps[:, None])
    return probs, ids.reshape(B)


# speedup vs baseline: 25.1443x; 1.1401x over previous
import functools

import jax
import jax.numpy as jnp
from jax import lax
from jax.experimental import pallas as pl
from jax.experimental.pallas import tpu as pltpu
from jax.experimental.pallas import tpu_sc as plsc

V = 100000
D = 2048
B = 64
VB = 2048
NB = (V + VB - 1) // VB
VPAD = NB * VB
BISECT_ROUNDS = 15
LO_BITS = 0x33000000
HI_BITS = 0x3F800000


def _bits_f32(i):
    import numpy as np
    return float(np.int32(i).view(np.float32))


def _quarter_mids(lo, hi):
    gap = hi - lo
    return lo + (gap >> 2), lo + (gap >> 1), hi - (gap >> 2)


_M1_0, _M2_0, _M3_0 = _quarter_mids(LO_BITS, HI_BITS)
_T1_0, _T2_0, _T3_0 = _bits_f32(_M1_0), _bits_f32(_M2_0), _bits_f32(_M3_0)

_RPW = 8
_NW_USED = B // _RPW


def _sc_gather(table, idx):
    info = plsc.get_sparse_core_info()
    nc = info.num_cores
    mesh = plsc.VectorSubcoreMesh(core_axis_name="c", subcore_axis_name="s")

    @functools.partial(
        pl.kernel,
        out_type=jax.ShapeDtypeStruct((B, D), jnp.float32),
        mesh=mesh,
        scratch_types=[
            pltpu.VMEM((_RPW,), jnp.int32),
            pltpu.VMEM((_RPW, D), jnp.float32),
            pltpu.SemaphoreType.DMA,
        ],
    )
    def k(table_hbm, idx_hbm, out_hbm, idx_v, rows_v, sem):
        wid = lax.axis_index("s") * nc + lax.axis_index("c")

        @pl.when(wid < _NW_USED)
        def _():
            base = wid * _RPW
            pltpu.sync_copy(idx_hbm.at[pl.ds(base, _RPW)], idx_v)
            pltpu.async_copy(table_hbm.at[idx_v], rows_v, sem).wait()
            pltpu.sync_copy(rows_v, out_hbm.at[pl.ds(base, _RPW)])

    return k(table, idx)


def _sampler_body(hs_ref, emb_ref, temp_ref, top_ref, out_ref, ids_ref,
                  xs_ref, m_ref, z_ref, thr_ref, den_ref, gmax_ref, gidx_ref):
    i = pl.program_id(0)

    @pl.when(i < NB)
    def _matmul():
        eb = emb_ref[...]
        x = lax.dot_general(hs_ref[...], eb,
                            (((1,), (1,)), ((), ())),
                            preferred_element_type=jnp.float32)
        x = x * (1.0 / temp_ref[...])
        col = i * VB + lax.broadcasted_iota(jnp.int32, (B, VB), 1)
        x = jnp.where(col < V, x, -1e30)
        xs_ref[:, pl.ds(pl.multiple_of(i * VB, VB), VB)] = x
        bm = jnp.max(x, axis=1, keepdims=True)
        m_ref[...] = jnp.where(i == 0, bm, jnp.maximum(m_ref[...], bm))

    @pl.when(i == NB)
    def _threshold():
        m = m_ref[...]
        zero = jnp.zeros((B, 1), jnp.float32)

        def expz(k, carry):
            z, s1, s2, s3 = carry
            sl = pl.ds(pl.multiple_of(k * VB, VB), VB)
            u = jnp.exp(xs_ref[:, sl] - m)
            xs_ref[:, sl] = u
            z = z + jnp.sum(u, axis=1, keepdims=True)
            s1 = s1 + jnp.sum(jnp.where(u > _T1_0, u, 0.0), axis=1, keepdims=True)
            s2 = s2 + jnp.sum(jnp.where(u > _T2_0, u, 0.0), axis=1, keepdims=True)
            s3 = s3 + jnp.sum(jnp.where(u > _T3_0, u, 0.0), axis=1, keepdims=True)
            return z, s1, s2, s3

        z, s1, s2, s3 = lax.fori_loop(0, NB, expz, (zero, zero, zero, zero))
        c = top_ref[...] * z

        def narrow(lo, hi, slo, m1, m2, m3, s1, s2, s3):
            p1, p2, p3 = s1 <= c, s2 <= c, s3 <= c
            hi_n = jnp.where(p1, m1, jnp.where(p2, m2, jnp.where(p3, m3, hi)))
            lo_n = jnp.where(p1, lo, jnp.where(p2, m1, jnp.where(p3, m2, m3)))
            slo_n = jnp.where(p1, slo, jnp.where(p2, s1, jnp.where(p3, s2, s3)))
            return lo_n, hi_n, slo_n

        lo0 = jnp.full((B, 1), LO_BITS, jnp.int32)
        hi0 = jnp.full((B, 1), HI_BITS, jnp.int32)
        m1_0 = jnp.full((B, 1), _M1_0, jnp.int32)
        m2_0 = jnp.full((B, 1), _M2_0, jnp.int32)
        m3_0 = jnp.full((B, 1), _M3_0, jnp.int32)
        lo, hi, slo = narrow(lo0, hi0, z, m1_0, m2_0, m3_0, s1, s2, s3)

        def bisect(_, carry):
            lo, hi, slo = carry
            gap = hi - lo
            m1 = lo + (gap >> 2)
            m2 = lo + (gap >> 1)
            m3 = hi - (gap >> 2)
            t1 = lax.bitcast_convert_type(m1, jnp.float32)
            t2 = lax.bitcast_convert_type(m2, jnp.float32)
            t3 = lax.bitcast_convert_type(m3, jnp.float32)

            def s_chunk(k, s):
                s1, s2, s3 = s
                u = xs_ref[:, pl.ds(pl.multiple_of(k * VB, VB), VB)]
                s1 = s1 + jnp.sum(jnp.where(u > t1, u, 0.0), axis=1, keepdims=True)
                s2 = s2 + jnp.sum(jnp.where(u > t2, u, 0.0), axis=1, keepdims=True)
                s3 = s3 + jnp.sum(jnp.where(u > t3, u, 0.0), axis=1, keepdims=True)
                return s1, s2, s3

            s1, s2, s3 = lax.fori_loop(0, NB, s_chunk, (zero, zero, zero))
            return narrow(lo, hi, slo, m1, m2, m3, s1, s2, s3)

        lo, hi, slo = lax.fori_loop(0, BISECT_ROUNDS, bisect, (lo, hi, slo))
        z_ref[...] = z
        thr_ref[...] = lax.bitcast_convert_type(hi, jnp.float32)
        den_ref[...] = slo / z

    @pl.when(i > NB)
    def _emit():
        blk = i - NB - 1
        u = xs_ref[:, pl.ds(pl.multiple_of(blk * VB, VB), VB)]
        p = u / z_ref[...]
        pf = jnp.where(u >= thr_ref[...], p / den_ref[...], 0.0)
        out_ref[...] = pf
        cm = jnp.max(pf, axis=1, keepdims=True)
        ci = jnp.argmax(pf, axis=1).astype(jnp.int32)[:, None] + blk * VB

        @pl.when(blk == 0)
        def _():
            gmax_ref[...] = cm
            gidx_ref[...] = ci

        @pl.when(blk > 0)
        def _():
            upd = cm > gmax_ref[...]
            gidx_ref[...] = jnp.where(upd, ci, gidx_ref[...])
            gmax_ref[...] = jnp.maximum(gmax_ref[...], cm)

        @pl.when(i == 2 * NB)
        def _():
            ids_ref[...] = gidx_ref[...]


_GRID = (2 * NB + 1,)
_IN_SPECS = [
    pl.BlockSpec((B, D), lambda i: (0, 0)),
    pl.BlockSpec((VB, D), lambda i: (jnp.minimum(i, NB - 1), 0)),
    pl.BlockSpec((B, 1), lambda i: (0, 0)),
    pl.BlockSpec((B, 1), lambda i: (0, 0)),
]
_OUT_SPECS = [
    pl.BlockSpec((B, VB), lambda i: (0, jnp.maximum(0, i - (NB + 1)))),
    pl.BlockSpec((B, 1), lambda i: (0, 0)),
]
_OUT_SHAPE = [
    jax.ShapeDtypeStruct((B, V), jnp.float32),
    jax.ShapeDtypeStruct((B, 1), jnp.int32),
]
_SCRATCH = [
    pltpu.VMEM((B, VPAD), jnp.float32),
    pltpu.VMEM((B, 1), jnp.float32),
    pltpu.VMEM((B, 1), jnp.float32),
    pltpu.VMEM((B, 1), jnp.float32),
    pltpu.VMEM((B, 1), jnp.float32),
    pltpu.VMEM((B, 1), jnp.float32),
    pltpu.VMEM((B, 1), jnp.int32),
]


def kernel(embedding, hidden_states, last_token_indices, temperatures, top_ps):
    hs = _sc_gather(hidden_states, last_token_indices.astype(jnp.int32))
    probs, ids = pl.pallas_call(
        _sampler_body,
        grid=_GRID,
        in_specs=_IN_SPECS,
        out_specs=_OUT_SPECS,
        out_shape=_OUT_SHAPE,
        scratch_shapes=_SCRATCH,
        compiler_params=pltpu.CompilerParams(vmem_limit_bytes=100 * 1024 * 1024),
    )(hs, embedding, temperatures[:, None], top_ps[:, None])
    return probs, ids.reshape(B)


# speedup vs baseline: 33.6204x; 1.3371x over previous
import functools

import jax
import jax.numpy as jnp
from jax import lax
from jax.experimental import pallas as pl
from jax.experimental.pallas import tpu as pltpu
from jax.experimental.pallas import tpu_sc as plsc

V = 100000
D = 2048
B = 64
VB = 2048
NB = (V + VB - 1) // VB
VPAD = NB * VB
BISECT_ROUNDS = 27
LO_BITS = 0x33000000
HI_BITS = 0x3F800000


def _bits_f32(i):
    import numpy as np
    return float(np.int32(i).view(np.float32))


def _quarter_mids(lo, hi):
    gap = hi - lo
    return lo + (gap >> 2), lo + (gap >> 1), hi - (gap >> 2)


_M1_0, _M2_0, _M3_0 = _quarter_mids(LO_BITS, HI_BITS)
_T1_0, _T2_0, _T3_0 = _bits_f32(_M1_0), _bits_f32(_M2_0), _bits_f32(_M3_0)

_RPW = 8
_NW_USED = B // _RPW


def _sc_gather(table, idx):
    info = plsc.get_sparse_core_info()
    nc = info.num_cores
    mesh = plsc.VectorSubcoreMesh(core_axis_name="c", subcore_axis_name="s")

    @functools.partial(
        pl.kernel,
        out_type=jax.ShapeDtypeStruct((B, D), jnp.float32),
        mesh=mesh,
        scratch_types=[
            pltpu.VMEM((_RPW,), jnp.int32),
            pltpu.VMEM((_RPW, D), jnp.float32),
            pltpu.SemaphoreType.DMA,
        ],
    )
    def k(table_hbm, idx_hbm, out_hbm, idx_v, rows_v, sem):
        wid = lax.axis_index("s") * nc + lax.axis_index("c")

        @pl.when(wid < _NW_USED)
        def _():
            base = wid * _RPW
            pltpu.sync_copy(idx_hbm.at[pl.ds(base, _RPW)], idx_v)
            pltpu.async_copy(table_hbm.at[idx_v], rows_v, sem).wait()
            pltpu.sync_copy(rows_v, out_hbm.at[pl.ds(base, _RPW)])

    return k(table, idx)


def _sampler_body(hs_ref, emb_ref, temp_ref, top_ref, out_ref, ids_ref,
                  xs_ref, m_ref, z_ref, thr_ref, den_ref, gmax_ref, gidx_ref):
    i = pl.program_id(0)

    @pl.when(i < NB)
    def _matmul():
        eb = emb_ref[...]
        x = lax.dot_general(hs_ref[...], eb,
                            (((1,), (1,)), ((), ())),
                            preferred_element_type=jnp.float32)
        x = x * (1.0 / temp_ref[...])
        col = i * VB + lax.broadcasted_iota(jnp.int32, (B, VB), 1)
        x = jnp.where(col < V, x, -1e30)
        xs_ref[:, pl.ds(pl.multiple_of(i * VB, VB), VB)] = x
        bm = jnp.max(x, axis=1, keepdims=True)
        m_ref[...] = jnp.where(i == 0, bm, jnp.maximum(m_ref[...], bm))

    @pl.when(i == NB)
    def _threshold():
        m = m_ref[...]
        zero = jnp.zeros((B, 1), jnp.float32)

        def expz(k, carry):
            z, s1, s2, s3 = carry
            sl = pl.ds(pl.multiple_of(k * VB, VB), VB)
            u = jnp.exp(xs_ref[:, sl] - m)
            xs_ref[:, sl] = u
            z = z + jnp.sum(u, axis=1, keepdims=True)
            s1 = s1 + jnp.sum(jnp.where(u > _T1_0, u, 0.0), axis=1, keepdims=True)
            s2 = s2 + jnp.sum(jnp.where(u > _T2_0, u, 0.0), axis=1, keepdims=True)
            s3 = s3 + jnp.sum(jnp.where(u > _T3_0, u, 0.0), axis=1, keepdims=True)
            return z, s1, s2, s3

        z, s1, s2, s3 = lax.fori_loop(0, NB, expz, (zero, zero, zero, zero))
        c = top_ref[...] * z

        def narrow(lo, hi, slo, m1, m2, m3, s1, s2, s3):
            p1, p2, p3 = s1 <= c, s2 <= c, s3 <= c
            hi_n = jnp.where(p1, m1, jnp.where(p2, m2, jnp.where(p3, m3, hi)))
            lo_n = jnp.where(p1, lo, jnp.where(p2, m1, jnp.where(p3, m2, m3)))
            slo_n = jnp.where(p1, slo, jnp.where(p2, s1, jnp.where(p3, s2, s3)))
            return lo_n, hi_n, slo_n

        lo0 = jnp.full((B, 1), LO_BITS, jnp.int32)
        hi0 = jnp.full((B, 1), HI_BITS, jnp.int32)
        m1_0 = jnp.full((B, 1), _M1_0, jnp.int32)
        m2_0 = jnp.full((B, 1), _M2_0, jnp.int32)
        m3_0 = jnp.full((B, 1), _M3_0, jnp.int32)
        lo, hi, slo = narrow(lo0, hi0, z, m1_0, m2_0, m3_0, s1, s2, s3)

        def bisect(_, carry):
            lo, hi, slo = carry
            mid = lo + ((hi - lo) >> 1)
            t = lax.bitcast_convert_type(mid, jnp.float32)
            u = xs_ref[...]
            s = jnp.sum(jnp.where(u > t, u, 0.0), axis=1, keepdims=True)
            pred = s <= c
            return (jnp.where(pred, lo, mid), jnp.where(pred, mid, hi),
                    jnp.where(pred, slo, s))

        lo, hi, slo = lax.fori_loop(0, BISECT_ROUNDS, bisect, (lo, hi, slo))
        z_ref[...] = z
        thr_ref[...] = lax.bitcast_convert_type(hi, jnp.float32)
        den_ref[...] = slo / z

    @pl.when(i > NB)
    def _emit():
        blk = i - NB - 1
        u = xs_ref[:, pl.ds(pl.multiple_of(blk * VB, VB), VB)]
        p = u / z_ref[...]
        pf = jnp.where(u >= thr_ref[...], p / den_ref[...], 0.0)
        out_ref[...] = pf
        cm = jnp.max(pf, axis=1, keepdims=True)
        ci = jnp.argmax(pf, axis=1).astype(jnp.int32)[:, None] + blk * VB

        @pl.when(blk == 0)
        def _():
            gmax_ref[...] = cm
            gidx_ref[...] = ci

        @pl.when(blk > 0)
        def _():
            upd = cm > gmax_ref[...]
            gidx_ref[...] = jnp.where(upd, ci, gidx_ref[...])
            gmax_ref[...] = jnp.maximum(gmax_ref[...], cm)

        @pl.when(i == 2 * NB)
        def _():
            ids_ref[...] = gidx_ref[...]


_GRID = (2 * NB + 1,)
_IN_SPECS = [
    pl.BlockSpec((B, D), lambda i: (0, 0)),
    pl.BlockSpec((VB, D), lambda i: (jnp.minimum(i, NB - 1), 0)),
    pl.BlockSpec((B, 1), lambda i: (0, 0)),
    pl.BlockSpec((B, 1), lambda i: (0, 0)),
]
_OUT_SPECS = [
    pl.BlockSpec((B, VB), lambda i: (0, jnp.maximum(0, i - (NB + 1)))),
    pl.BlockSpec((B, 1), lambda i: (0, 0)),
]
_OUT_SHAPE = [
    jax.ShapeDtypeStruct((B, V), jnp.float32),
    jax.ShapeDtypeStruct((B, 1), jnp.int32),
]
_SCRATCH = [
    pltpu.VMEM((B, VPAD), jnp.float32),
    pltpu.VMEM((B, 1), jnp.float32),
    pltpu.VMEM((B, 1), jnp.float32),
    pltpu.VMEM((B, 1), jnp.float32),
    pltpu.VMEM((B, 1), jnp.float32),
    pltpu.VMEM((B, 1), jnp.float32),
    pltpu.VMEM((B, 1), jnp.int32),
]


def kernel(embedding, hidden_states, last_token_indices, temperatures, top_ps):
    hs = _sc_gather(hidden_states, last_token_indices.astype(jnp.int32))
    probs, ids = pl.pallas_call(
        _sampler_body,
        grid=_GRID,
        in_specs=_IN_SPECS,
        out_specs=_OUT_SPECS,
        out_shape=_OUT_SHAPE,
        scratch_shapes=_SCRATCH,
        compiler_params=pltpu.CompilerParams(vmem_limit_bytes=120 * 1024 * 1024),
    )(hs, embedding, temperatures[:, None], top_ps[:, None])
    return probs, ids.reshape(B)


# speedup vs baseline: 34.3921x; 1.0230x over previous
import functools

import jax
import jax.numpy as jnp
from jax import lax
from jax.experimental import pallas as pl
from jax.experimental.pallas import tpu as pltpu
from jax.experimental.pallas import tpu_sc as plsc

V = 100000
D = 2048
B = 64
VB = 2048
NB = (V + VB - 1) // VB
VPAD = NB * VB
BISECT_ROUNDS = 26
LO_BITS = 0x33000000
HI_BITS = 0x3F800000


def _bits_f32(i):
    import numpy as np
    return float(np.int32(i).view(np.float32))


def _quarter_mids(lo, hi):
    gap = hi - lo
    return lo + (gap >> 2), lo + (gap >> 1), hi - (gap >> 2)


_M1_0, _M2_0, _M3_0 = _quarter_mids(LO_BITS, HI_BITS)
_T1_0, _T2_0, _T3_0 = _bits_f32(_M1_0), _bits_f32(_M2_0), _bits_f32(_M3_0)

_RPW = 8
_NW_USED = B // _RPW


def _sc_gather(table, idx):
    info = plsc.get_sparse_core_info()
    nc = info.num_cores
    mesh = plsc.VectorSubcoreMesh(core_axis_name="c", subcore_axis_name="s")

    @functools.partial(
        pl.kernel,
        out_type=jax.ShapeDtypeStruct((B, D), jnp.float32),
        mesh=mesh,
        scratch_types=[
            pltpu.VMEM((_RPW,), jnp.int32),
            pltpu.VMEM((_RPW, D), jnp.float32),
            pltpu.SemaphoreType.DMA,
        ],
    )
    def k(table_hbm, idx_hbm, out_hbm, idx_v, rows_v, sem):
        wid = lax.axis_index("s") * nc + lax.axis_index("c")

        @pl.when(wid < _NW_USED)
        def _():
            base = wid * _RPW
            pltpu.sync_copy(idx_hbm.at[pl.ds(base, _RPW)], idx_v)
            pltpu.async_copy(table_hbm.at[idx_v], rows_v, sem).wait()
            pltpu.sync_copy(rows_v, out_hbm.at[pl.ds(base, _RPW)])

    return k(table, idx)


def _sampler_body(hs_ref, emb_ref, temp_ref, top_ref, out_ref, ids_ref,
                  xs_ref, m_ref, z_ref, thr_ref, den_ref, gmax_ref, gidx_ref):
    i = pl.program_id(0)

    @pl.when(i < NB)
    def _matmul():
        eb = emb_ref[...]
        x = lax.dot_general(hs_ref[...], eb,
                            (((1,), (1,)), ((), ())),
                            preferred_element_type=jnp.float32)
        x = x * (1.0 / temp_ref[...])
        col = i * VB + lax.broadcasted_iota(jnp.int32, (B, VB), 1)
        x = jnp.where(col < V, x, -1e30)
        xs_ref[:, pl.ds(pl.multiple_of(i * VB, VB), VB)] = x
        bm = jnp.max(x, axis=1, keepdims=True)
        m_ref[...] = jnp.where(i == 0, bm, jnp.maximum(m_ref[...], bm))

    @pl.when(i == NB)
    def _threshold():
        m = m_ref[...]
        zero = jnp.zeros((B, 1), jnp.float32)

        ecb = VPAD // 14

        def expz(k, carry):
            z, s1, s2, s3 = carry
            sl = pl.ds(pl.multiple_of(k * ecb, ecb), ecb)
            u = jnp.exp(xs_ref[:, sl] - m)
            xs_ref[:, sl] = u
            z = z + jnp.sum(u, axis=1, keepdims=True)
            s1 = s1 + jnp.sum(jnp.where(u > _T1_0, u, 0.0), axis=1, keepdims=True)
            s2 = s2 + jnp.sum(jnp.where(u > _T2_0, u, 0.0), axis=1, keepdims=True)
            s3 = s3 + jnp.sum(jnp.where(u > _T3_0, u, 0.0), axis=1, keepdims=True)
            return z, s1, s2, s3

        z, s1, s2, s3 = lax.fori_loop(0, 14, expz, (zero, zero, zero, zero))
        c = top_ref[...] * z

        def narrow(lo, hi, slo, m1, m2, m3, s1, s2, s3):
            p1, p2, p3 = s1 <= c, s2 <= c, s3 <= c
            hi_n = jnp.where(p1, m1, jnp.where(p2, m2, jnp.where(p3, m3, hi)))
            lo_n = jnp.where(p1, lo, jnp.where(p2, m1, jnp.where(p3, m2, m3)))
            slo_n = jnp.where(p1, slo, jnp.where(p2, s1, jnp.where(p3, s2, s3)))
            return lo_n, hi_n, slo_n

        lo0 = jnp.full((B, 1), LO_BITS, jnp.int32)
        hi0 = jnp.full((B, 1), HI_BITS, jnp.int32)
        m1_0 = jnp.full((B, 1), _M1_0, jnp.int32)
        m2_0 = jnp.full((B, 1), _M2_0, jnp.int32)
        m3_0 = jnp.full((B, 1), _M3_0, jnp.int32)
        lo, hi, slo = narrow(lo0, hi0, z, m1_0, m2_0, m3_0, s1, s2, s3)

        def bisect(_, carry):
            lo, hi, slo = carry
            mid = lo + ((hi - lo) >> 1)
            t = lax.bitcast_convert_type(mid, jnp.float32)
            u = xs_ref[...]
            s = jnp.sum(jnp.where(u > t, u, 0.0), axis=1, keepdims=True)
            pred = s <= c
            return (jnp.where(pred, lo, mid), jnp.where(pred, mid, hi),
                    jnp.where(pred, slo, s))

        lo, hi, slo = lax.fori_loop(0, BISECT_ROUNDS, bisect, (lo, hi, slo))
        z_ref[...] = z
        thr_ref[...] = lax.bitcast_convert_type(hi, jnp.float32)
        den_ref[...] = slo / z

    @pl.when(i > NB)
    def _emit():
        blk = i - NB - 1
        u = xs_ref[:, pl.ds(pl.multiple_of(blk * VB, VB), VB)]
        r = 1.0 / (z_ref[...] * den_ref[...])
        pf = jnp.where(u >= thr_ref[...], u * r, 0.0)
        out_ref[...] = pf
        cm = jnp.max(pf, axis=1, keepdims=True)
        ci = jnp.argmax(pf, axis=1).astype(jnp.int32)[:, None] + blk * VB

        @pl.when(blk == 0)
        def _():
            gmax_ref[...] = cm
            gidx_ref[...] = ci

        @pl.when(blk > 0)
        def _():
            upd = cm > gmax_ref[...]
            gidx_ref[...] = jnp.where(upd, ci, gidx_ref[...])
            gmax_ref[...] = jnp.maximum(gmax_ref[...], cm)

        @pl.when(i == 2 * NB)
        def _():
            ids_ref[...] = gidx_ref[...]


_GRID = (2 * NB + 1,)
_IN_SPECS = [
    pl.BlockSpec((B, D), lambda i: (0, 0)),
    pl.BlockSpec((VB, D), lambda i: (jnp.minimum(i, NB - 1), 0)),
    pl.BlockSpec((B, 1), lambda i: (0, 0)),
    pl.BlockSpec((B, 1), lambda i: (0, 0)),
]
_OUT_SPECS = [
    pl.BlockSpec((B, VB), lambda i: (0, jnp.maximum(0, i - (NB + 1)))),
    pl.BlockSpec((B, 1), lambda i: (0, 0)),
]
_OUT_SHAPE = [
    jax.ShapeDtypeStruct((B, V), jnp.float32),
    jax.ShapeDtypeStruct((B, 1), jnp.int32),
]
_SCRATCH = [
    pltpu.VMEM((B, VPAD), jnp.float32),
    pltpu.VMEM((B, 1), jnp.float32),
    pltpu.VMEM((B, 1), jnp.float32),
    pltpu.VMEM((B, 1), jnp.float32),
    pltpu.VMEM((B, 1), jnp.float32),
    pltpu.VMEM((B, 1), jnp.float32),
    pltpu.VMEM((B, 1), jnp.int32),
]


def kernel(embedding, hidden_states, last_token_indices, temperatures, top_ps):
    hs = _sc_gather(hidden_states, last_token_indices.astype(jnp.int32))
    probs, ids = pl.pallas_call(
        _sampler_body,
        grid=_GRID,
        in_specs=_IN_SPECS,
        out_specs=_OUT_SPECS,
        out_shape=_OUT_SHAPE,
        scratch_shapes=_SCRATCH,
        compiler_params=pltpu.CompilerParams(vmem_limit_bytes=120 * 1024 * 1024),
    )(hs, embedding, temperatures[:, None], top_ps[:, None])
    return probs, ids.reshape(B)


# speedup vs baseline: 34.8229x; 1.0125x over previous
import functools

import jax
import jax.numpy as jnp
from jax import lax
from jax.experimental import pallas as pl
from jax.experimental.pallas import tpu as pltpu
from jax.experimental.pallas import tpu_sc as plsc

V = 100000
D = 2048
B = 64
VB = 2048
NB = (V + VB - 1) // VB
VPAD = NB * VB
BISECT_ROUNDS = 26
LO_BITS = 0x33000000
HI_BITS = 0x3F800000


def _bits_f32(i):
    import numpy as np
    return float(np.int32(i).view(np.float32))


def _quarter_mids(lo, hi):
    gap = hi - lo
    return lo + (gap >> 2), lo + (gap >> 1), hi - (gap >> 2)


_M1_0, _M2_0, _M3_0 = _quarter_mids(LO_BITS, HI_BITS)
_T1_0, _T2_0, _T3_0 = _bits_f32(_M1_0), _bits_f32(_M2_0), _bits_f32(_M3_0)

_RPW = 8
_NW_USED = B // _RPW


def _sc_gather(table, idx):
    info = plsc.get_sparse_core_info()
    nc = info.num_cores
    mesh = plsc.VectorSubcoreMesh(core_axis_name="c", subcore_axis_name="s")

    @functools.partial(
        pl.kernel,
        out_type=jax.ShapeDtypeStruct((B, D), jnp.float32),
        mesh=mesh,
        scratch_types=[
            pltpu.VMEM((_RPW,), jnp.int32),
            pltpu.VMEM((_RPW, D), jnp.float32),
            pltpu.SemaphoreType.DMA,
        ],
    )
    def k(table_hbm, idx_hbm, out_hbm, idx_v, rows_v, sem):
        wid = lax.axis_index("s") * nc + lax.axis_index("c")

        @pl.when(wid < _NW_USED)
        def _():
            base = wid * _RPW
            pltpu.sync_copy(idx_hbm.at[pl.ds(base, _RPW)], idx_v)
            pltpu.async_copy(table_hbm.at[idx_v], rows_v, sem).wait()
            pltpu.sync_copy(rows_v, out_hbm.at[pl.ds(base, _RPW)])

    return k(table, idx)


def _sampler_body(hs_ref, emb_ref, temp_ref, top_ref, out_ref, ids_ref,
                  xs_ref, m_ref, z_ref, thr_ref, den_ref, gidx_ref):
    i = pl.program_id(0)

    @pl.when(i < NB)
    def _matmul():
        eb = emb_ref[...]
        x = lax.dot_general(hs_ref[...], eb,
                            (((1,), (1,)), ((), ())),
                            preferred_element_type=jnp.float32)
        x = x * (1.0 / temp_ref[...])
        col = i * VB + lax.broadcasted_iota(jnp.int32, (B, VB), 1)
        x = jnp.where(col < V, x, -1e30)
        xs_ref[:, pl.ds(pl.multiple_of(i * VB, VB), VB)] = x
        bm = jnp.max(x, axis=1, keepdims=True)
        bi = jnp.argmax(x, axis=1).astype(jnp.int32)[:, None] + i * VB

        @pl.when(i == 0)
        def _():
            m_ref[...] = bm
            gidx_ref[...] = bi

        @pl.when(i > 0)
        def _():
            upd = bm > m_ref[...]
            gidx_ref[...] = jnp.where(upd, bi, gidx_ref[...])
            m_ref[...] = jnp.maximum(m_ref[...], bm)

    @pl.when(i == NB)
    def _threshold():
        m = m_ref[...]
        zero = jnp.zeros((B, 1), jnp.float32)

        ecb = VPAD // 14

        def expz(k, carry):
            z, s1, s2, s3 = carry
            sl = pl.ds(pl.multiple_of(k * ecb, ecb), ecb)
            u = jnp.exp(xs_ref[:, sl] - m)
            xs_ref[:, sl] = u
            z = z + jnp.sum(u, axis=1, keepdims=True)
            s1 = s1 + jnp.sum(jnp.where(u > _T1_0, u, 0.0), axis=1, keepdims=True)
            s2 = s2 + jnp.sum(jnp.where(u > _T2_0, u, 0.0), axis=1, keepdims=True)
            s3 = s3 + jnp.sum(jnp.where(u > _T3_0, u, 0.0), axis=1, keepdims=True)
            return z, s1, s2, s3

        z, s1, s2, s3 = lax.fori_loop(0, 14, expz, (zero, zero, zero, zero))
        c = top_ref[...] * z

        def narrow(lo, hi, slo, m1, m2, m3, s1, s2, s3):
            p1, p2, p3 = s1 <= c, s2 <= c, s3 <= c
            hi_n = jnp.where(p1, m1, jnp.where(p2, m2, jnp.where(p3, m3, hi)))
            lo_n = jnp.where(p1, lo, jnp.where(p2, m1, jnp.where(p3, m2, m3)))
            slo_n = jnp.where(p1, slo, jnp.where(p2, s1, jnp.where(p3, s2, s3)))
            return lo_n, hi_n, slo_n

        lo0 = jnp.full((B, 1), LO_BITS, jnp.int32)
        hi0 = jnp.full((B, 1), HI_BITS, jnp.int32)
        m1_0 = jnp.full((B, 1), _M1_0, jnp.int32)
        m2_0 = jnp.full((B, 1), _M2_0, jnp.int32)
        m3_0 = jnp.full((B, 1), _M3_0, jnp.int32)
        lo, hi, slo = narrow(lo0, hi0, z, m1_0, m2_0, m3_0, s1, s2, s3)

        def bisect(_, carry):
            lo, hi, slo = carry
            mid = lo + ((hi - lo) >> 1)
            t = lax.bitcast_convert_type(mid, jnp.float32)
            u = xs_ref[...]
            s = jnp.sum(jnp.where(u > t, u, 0.0), axis=1, keepdims=True)
            pred = s <= c
            return (jnp.where(pred, lo, mid), jnp.where(pred, mid, hi),
                    jnp.where(pred, slo, s))

        lo, hi, slo = lax.fori_loop(0, BISECT_ROUNDS, bisect, (lo, hi, slo))
        z_ref[...] = z
        thr_ref[...] = lax.bitcast_convert_type(hi, jnp.float32)
        den_ref[...] = slo / z

    @pl.when(i > NB)
    def _emit():
        blk = i - NB - 1
        u = xs_ref[:, pl.ds(pl.multiple_of(blk * VB, VB), VB)]
        r = 1.0 / (z_ref[...] * den_ref[...])
        out_ref[...] = jnp.where(u >= thr_ref[...], u * r, 0.0)

        @pl.when(i == 2 * NB)
        def _():
            ids_ref[...] = gidx_ref[...]


_GRID = (2 * NB + 1,)
_IN_SPECS = [
    pl.BlockSpec((B, D), lambda i: (0, 0)),
    pl.BlockSpec((VB, D), lambda i: (jnp.minimum(i, NB - 1), 0)),
    pl.BlockSpec((B, 1), lambda i: (0, 0)),
    pl.BlockSpec((B, 1), lambda i: (0, 0)),
]
_OUT_SPECS = [
    pl.BlockSpec((B, VB), lambda i: (0, jnp.maximum(0, i - (NB + 1)))),
    pl.BlockSpec((B, 1), lambda i: (0, 0)),
]
_OUT_SHAPE = [
    jax.ShapeDtypeStruct((B, V), jnp.float32),
    jax.ShapeDtypeStruct((B, 1), jnp.int32),
]
_SCRATCH = [
    pltpu.VMEM((B, VPAD), jnp.float32),
    pltpu.VMEM((B, 1), jnp.float32),
    pltpu.VMEM((B, 1), jnp.float32),
    pltpu.VMEM((B, 1), jnp.float32),
    pltpu.VMEM((B, 1), jnp.float32),
    pltpu.VMEM((B, 1), jnp.int32),
]


def kernel(embedding, hidden_states, last_token_indices, temperatures, top_ps):
    hs = _sc_gather(hidden_states, last_token_indices.astype(jnp.int32))
    probs, ids = pl.pallas_call(
        _sampler_body,
        grid=_GRID,
        in_specs=_IN_SPECS,
        out_specs=_OUT_SPECS,
        out_shape=_OUT_SHAPE,
        scratch_shapes=_SCRATCH,
        compiler_params=pltpu.CompilerParams(vmem_limit_bytes=120 * 1024 * 1024),
    )(hs, embedding, temperatures[:, None], top_ps[:, None])
    return probs, ids.reshape(B)


# speedup vs baseline: 34.8375x; 1.0004x over previous
import functools

import jax
import jax.numpy as jnp
from jax import lax
from jax.experimental import pallas as pl
from jax.experimental.pallas import tpu as pltpu
from jax.experimental.pallas import tpu_sc as plsc

V = 100000
D = 2048
B = 64
VB = 2048
NB = (V + VB - 1) // VB
VPAD = NB * VB
BISECT_ROUNDS = 26
LO_BITS = 0x33000000
HI_BITS = 0x3F800000


def _bits_f32(i):
    import numpy as np
    return float(np.int32(i).view(np.float32))


def _quarter_mids(lo, hi):
    gap = hi - lo
    return lo + (gap >> 2), lo + (gap >> 1), hi - (gap >> 2)


_M1_0, _M2_0, _M3_0 = _quarter_mids(LO_BITS, HI_BITS)
_T1_0, _T2_0, _T3_0 = _bits_f32(_M1_0), _bits_f32(_M2_0), _bits_f32(_M3_0)

_RPW = 8
_NW_USED = B // _RPW


def _sc_gather(table, idx):
    info = plsc.get_sparse_core_info()
    nc = info.num_cores
    mesh = plsc.VectorSubcoreMesh(core_axis_name="c", subcore_axis_name="s")

    @functools.partial(
        pl.kernel,
        out_type=jax.ShapeDtypeStruct((B, D), jnp.float32),
        mesh=mesh,
        scratch_types=[
            pltpu.VMEM((_RPW,), jnp.int32),
            pltpu.VMEM((_RPW, D), jnp.float32),
            pltpu.SemaphoreType.DMA,
        ],
    )
    def k(table_hbm, idx_hbm, out_hbm, idx_v, rows_v, sem):
        wid = lax.axis_index("s") * nc + lax.axis_index("c")

        @pl.when(wid < _NW_USED)
        def _():
            base = wid * _RPW
            pltpu.sync_copy(idx_hbm.at[pl.ds(base, _RPW)], idx_v)
            pltpu.async_copy(table_hbm.at[idx_v], rows_v, sem).wait()
            pltpu.sync_copy(rows_v, out_hbm.at[pl.ds(base, _RPW)])

    return k(table, idx)


def _sampler_body(hs_ref, emb_ref, temp_ref, top_ref, out_ref, ids_ref,
                  xs_ref, m_ref, z_ref, thr_ref, den_ref, gidx_ref):
    i = pl.program_id(0)

    @pl.when(i < NB)
    def _matmul():
        x = lax.dot_general(hs_ref[...], emb_ref[...],
                            (((1,), (1,)), ((), ())),
                            preferred_element_type=jnp.float32)
        x = x * (1.0 / temp_ref[...])
        col = i * VB + lax.broadcasted_iota(jnp.int32, (B, VB), 1)
        x = jnp.where(col < V, x, -1e30)
        xs_ref[:, pl.ds(pl.multiple_of(i * VB, VB), VB)] = x
        bm = jnp.max(x, axis=1, keepdims=True)
        bi = jnp.argmax(x, axis=1).astype(jnp.int32)[:, None] + i * VB

        @pl.when(i == 0)
        def _():
            m_ref[...] = bm
            gidx_ref[...] = bi

        @pl.when(i > 0)
        def _():
            upd = bm > m_ref[...]
            gidx_ref[...] = jnp.where(upd, bi, gidx_ref[...])
            m_ref[...] = jnp.maximum(m_ref[...], bm)

    @pl.when(i == NB)
    def _threshold():
        m = m_ref[...]
        zero = jnp.zeros((B, 1), jnp.float32)

        ecb = VPAD // 14

        def expz(k, carry):
            z, s1, s2, s3 = carry
            sl = pl.ds(pl.multiple_of(k * ecb, ecb), ecb)
            u = jnp.exp(xs_ref[:, sl] - m)
            xs_ref[:, sl] = u
            z = z + jnp.sum(u, axis=1, keepdims=True)
            s1 = s1 + jnp.sum(jnp.where(u > _T1_0, u, 0.0), axis=1, keepdims=True)
            s2 = s2 + jnp.sum(jnp.where(u > _T2_0, u, 0.0), axis=1, keepdims=True)
            s3 = s3 + jnp.sum(jnp.where(u > _T3_0, u, 0.0), axis=1, keepdims=True)
            return z, s1, s2, s3

        z, s1, s2, s3 = lax.fori_loop(0, 14, expz, (zero, zero, zero, zero))
        c = top_ref[...] * z

        def narrow(lo, hi, slo, m1, m2, m3, s1, s2, s3):
            p1, p2, p3 = s1 <= c, s2 <= c, s3 <= c
            hi_n = jnp.where(p1, m1, jnp.where(p2, m2, jnp.where(p3, m3, hi)))
            lo_n = jnp.where(p1, lo, jnp.where(p2, m1, jnp.where(p3, m2, m3)))
            slo_n = jnp.where(p1, slo, jnp.where(p2, s1, jnp.where(p3, s2, s3)))
            return lo_n, hi_n, slo_n

        lo0 = jnp.full((B, 1), LO_BITS, jnp.int32)
        hi0 = jnp.full((B, 1), HI_BITS, jnp.int32)
        m1_0 = jnp.full((B, 1), _M1_0, jnp.int32)
        m2_0 = jnp.full((B, 1), _M2_0, jnp.int32)
        m3_0 = jnp.full((B, 1), _M3_0, jnp.int32)
        lo, hi, slo = narrow(lo0, hi0, z, m1_0, m2_0, m3_0, s1, s2, s3)

        def bisect(_, carry):
            lo, hi, slo = carry
            mid = lo + ((hi - lo) >> 1)
            t = lax.bitcast_convert_type(mid, jnp.float32)
            u = xs_ref[...]
            s = jnp.sum(jnp.where(u > t, u, 0.0), axis=1, keepdims=True)
            pred = s <= c
            return (jnp.where(pred, lo, mid), jnp.where(pred, mid, hi),
                    jnp.where(pred, slo, s))

        lo, hi, slo = lax.fori_loop(0, BISECT_ROUNDS, bisect, (lo, hi, slo))
        z_ref[...] = z
        thr_ref[...] = lax.bitcast_convert_type(hi, jnp.float32)
        den_ref[...] = slo / z

    @pl.when(i > NB)
    def _emit():
        blk = i - NB - 1
        u = xs_ref[:, pl.ds(pl.multiple_of(blk * VB, VB), VB)]
        r = 1.0 / (z_ref[...] * den_ref[...])
        out_ref[...] = jnp.where(u >= thr_ref[...], u * r, 0.0)

        @pl.when(i == 2 * NB)
        def _():
            ids_ref[...] = gidx_ref[...]


_GRID = (2 * NB + 1,)
_IN_SPECS = [
    pl.BlockSpec((B, D), lambda i: (0, 0)),
    pl.BlockSpec((VB, D), lambda i: (jnp.minimum(i, NB - 1), 0)),
    pl.BlockSpec((B, 1), lambda i: (0, 0)),
    pl.BlockSpec((B, 1), lambda i: (0, 0)),
]
_OUT_SPECS = [
    pl.BlockSpec((B, VB), lambda i: (0, jnp.maximum(0, i - (NB + 1)))),
    pl.BlockSpec((B, 1), lambda i: (0, 0)),
]
_OUT_SHAPE = [
    jax.ShapeDtypeStruct((B, V), jnp.float32),
    jax.ShapeDtypeStruct((B, 1), jnp.int32),
]
_SCRATCH = [
    pltpu.VMEM((B, VPAD), jnp.float32),
    pltpu.VMEM((B, 1), jnp.float32),
    pltpu.VMEM((B, 1), jnp.float32),
    pltpu.VMEM((B, 1), jnp.float32),
    pltpu.VMEM((B, 1), jnp.float32),
    pltpu.VMEM((B, 1), jnp.int32),
]


def kernel(embedding, hidden_states, last_token_indices, temperatures, top_ps):
    hs = _sc_gather(hidden_states, last_token_indices.astype(jnp.int32))
    probs, ids = pl.pallas_call(
        _sampler_body,
        grid=_GRID,
        in_specs=_IN_SPECS,
        out_specs=_OUT_SPECS,
        out_shape=_OUT_SHAPE,
        scratch_shapes=_SCRATCH,
        compiler_params=pltpu.CompilerParams(vmem_limit_bytes=120 * 1024 * 1024),
    )(hs, embedding, temperatures[:, None], top_ps[:, None])
    return probs, ids.reshape(B)
